# Initial kernel scaffold; baseline (speedup 1.0000x reference)
import jax
import jax.numpy as jnp
from jax.experimental import pallas as pl


def kernel(word_inputs, feature_inputs_0, feature_inputs_1, word_seq_lengths, char_inputs, char_seq_lengths, char_seq_recover, word_table, feat_table_0, feat_table_1):
    raise NotImplementedError("write your pallas kernel here")


# speedup vs baseline: 2.2327x; 2.2327x over previous
import functools

import jax
import jax.numpy as jnp
from jax import lax
from jax.experimental import pallas as pl
from jax.experimental.pallas import tpu as pltpu
from jax.experimental.pallas import tpu_sc as plsc

VOCAB = 1000000
EMB = 64
FVOCAB = 100000
FEMB = 16
B = 4096
L = 50

NC = 2
NS = 16
NW = NC * NS
N = B * L
N_PER_W = N // NW
CHUNK = 128
NCHUNK = N_PER_W // CHUNK
OUT_D = EMB + 2 * FEMB


def _sc_gather_concat():
    mesh = plsc.VectorSubcoreMesh(core_axis_name="c", subcore_axis_name="s")

    @functools.partial(
        pl.kernel,
        out_type=jax.ShapeDtypeStruct((N, OUT_D), jnp.float32),
        mesh=mesh,
        compiler_params=pltpu.CompilerParams(use_tc_tiling_on_sc=False),
        scratch_types=[
            pltpu.VMEM((NCHUNK, CHUNK), jnp.int32),
            pltpu.VMEM((NCHUNK, CHUNK), jnp.int32),
            pltpu.VMEM((NCHUNK, CHUNK), jnp.int32),
            pltpu.VMEM((CHUNK, EMB), jnp.float32),
            pltpu.VMEM((CHUNK, FEMB), jnp.float32),
            pltpu.VMEM((CHUNK, FEMB), jnp.float32),
            pltpu.SemaphoreType.DMA,
            pltpu.SemaphoreType.DMA,
            pltpu.SemaphoreType.DMA,
        ],
    )
    def k(widx_hbm, f0idx_hbm, f1idx_hbm, wtab_hbm, f0tab_hbm, f1tab_hbm,
          out_hbm, widx_v, f0idx_v, f1idx_v, wrows, f0rows, f1rows,
          sem_w, sem_0, sem_1):
        wid = lax.axis_index("s") * NC + lax.axis_index("c")
        base = wid * N_PER_W
        pltpu.sync_copy(widx_hbm.at[wid], widx_v)
        pltpu.sync_copy(f0idx_hbm.at[wid], f0idx_v)
        pltpu.sync_copy(f1idx_hbm.at[wid], f1idx_v)

        def step(j, carry):
            cw = pltpu.async_copy(wtab_hbm.at[widx_v.at[j]], wrows, sem_w)
            c0 = pltpu.async_copy(f0tab_hbm.at[f0idx_v.at[j]], f0rows, sem_0)
            c1 = pltpu.async_copy(f1tab_hbm.at[f1idx_v.at[j]], f1rows, sem_1)
            cw.wait()
            c0.wait()
            c1.wait()
            row0 = base + j * CHUNK
            pltpu.sync_copy(wrows, out_hbm.at[pl.ds(row0, CHUNK), pl.ds(0, EMB)])
            pltpu.sync_copy(f0rows, out_hbm.at[pl.ds(row0, CHUNK), pl.ds(EMB, FEMB)])
            pltpu.sync_copy(f1rows, out_hbm.at[pl.ds(row0, CHUNK), pl.ds(EMB + FEMB, FEMB)])
            return carry

        lax.fori_loop(0, NCHUNK, step, 0)

    return k


_GATHER = _sc_gather_concat()


def kernel(word_inputs, feature_inputs_0, feature_inputs_1, word_seq_lengths,
           char_inputs, char_seq_lengths, char_seq_recover,
           word_table, feat_table_0, feat_table_1):
    widx = jnp.reshape(word_inputs.astype(jnp.int32), (NW, NCHUNK, CHUNK))
    f0idx = jnp.reshape(feature_inputs_0.astype(jnp.int32), (NW, NCHUNK, CHUNK))
    f1idx = jnp.reshape(feature_inputs_1.astype(jnp.int32), (NW, NCHUNK, CHUNK))
    out = _GATHER(widx, f0idx, f1idx, word_table, feat_table_0, feat_table_1)
    return jnp.reshape(out, (B, L, OUT_D))


# speedup vs baseline: 2.3217x; 1.0399x over previous
import functools

import jax
import jax.numpy as jnp
from jax import lax
from jax.experimental import pallas as pl
from jax.experimental.pallas import tpu as pltpu
from jax.experimental.pallas import tpu_sc as plsc

VOCAB = 1000000
EMB = 64
FVOCAB = 100000
FEMB = 16
B = 4096
L = 50

NC = 2
NS = 16
NW = NC * NS
N = B * L
N_PER_W = N // NW
CHUNK = 128
NCHUNK = N_PER_W // CHUNK
OUT_D = EMB + 2 * FEMB


def _sc_gather_concat():
    mesh = plsc.VectorSubcoreMesh(core_axis_name="c", subcore_axis_name="s")

    @functools.partial(
        pl.kernel,
        out_type=jax.ShapeDtypeStruct((N, OUT_D), jnp.float32),
        mesh=mesh,
        compiler_params=pltpu.CompilerParams(use_tc_tiling_on_sc=False),
        scratch_types=[
            pltpu.VMEM((NCHUNK, CHUNK), jnp.int32),
            pltpu.VMEM((NCHUNK, CHUNK), jnp.int32),
            pltpu.VMEM((NCHUNK, CHUNK), jnp.int32),
            pltpu.VMEM((2, CHUNK, EMB), jnp.float32),
            pltpu.VMEM((2, CHUNK, FEMB), jnp.float32),
            pltpu.VMEM((2, CHUNK, FEMB), jnp.float32),
            pltpu.SemaphoreType.DMA,
            pltpu.SemaphoreType.DMA,
            pltpu.SemaphoreType.DMA,
            pltpu.SemaphoreType.DMA,
            pltpu.SemaphoreType.DMA,
            pltpu.SemaphoreType.DMA,
        ],
    )
    def k(widx_hbm, f0idx_hbm, f1idx_hbm, wtab_hbm, f0tab_hbm, f1tab_hbm,
          out_hbm, widx_v, f0idx_v, f1idx_v, wrows, f0rows, f1rows,
          sem_gw, sem_g0, sem_g1, sem_ww, sem_w0, sem_w1):
        wid = lax.axis_index("s") * NC + lax.axis_index("c")
        base = wid * N_PER_W
        pltpu.sync_copy(widx_hbm.at[wid], widx_v)
        pltpu.sync_copy(f0idx_hbm.at[wid], f0idx_v)
        pltpu.sync_copy(f1idx_hbm.at[wid], f1idx_v)

        def gathers(j, s):
            pltpu.async_copy(wtab_hbm.at[widx_v.at[j]], wrows.at[s], sem_gw)
            pltpu.async_copy(f0tab_hbm.at[f0idx_v.at[j]], f0rows.at[s], sem_g0)
            pltpu.async_copy(f1tab_hbm.at[f1idx_v.at[j]], f1rows.at[s], sem_g1)

        def out_slices(j):
            row0 = base + j * CHUNK
            return (out_hbm.at[pl.ds(row0, CHUNK), pl.ds(0, EMB)],
                    out_hbm.at[pl.ds(row0, CHUNK), pl.ds(EMB, FEMB)],
                    out_hbm.at[pl.ds(row0, CHUNK), pl.ds(EMB + FEMB, FEMB)])

        def wait_writes(j, s):
            ow, o0, o1 = out_slices(j)
            pltpu.make_async_copy(wrows.at[s], ow, sem_ww).wait()
            pltpu.make_async_copy(f0rows.at[s], o0, sem_w0).wait()
            pltpu.make_async_copy(f1rows.at[s], o1, sem_w1).wait()

        gathers(0, 0)

        def step(j, carry):
            s = lax.rem(j, 2)
            pl.when(j >= 1)(lambda: wait_writes(j - 1, 1 - s))
            pl.when(j + 1 < NCHUNK)(lambda: gathers(j + 1, 1 - s))
            pltpu.make_async_copy(wtab_hbm.at[widx_v.at[j]], wrows.at[s], sem_gw).wait()
            pltpu.make_async_copy(f0tab_hbm.at[f0idx_v.at[j]], f0rows.at[s], sem_g0).wait()
            pltpu.make_async_copy(f1tab_hbm.at[f1idx_v.at[j]], f1rows.at[s], sem_g1).wait()
            ow, o0, o1 = out_slices(j)
            pltpu.async_copy(wrows.at[s], ow, sem_ww)
            pltpu.async_copy(f0rows.at[s], o0, sem_w0)
            pltpu.async_copy(f1rows.at[s], o1, sem_w1)
            return carry

        lax.fori_loop(0, NCHUNK, step, 0)
        wait_writes(NCHUNK - 1, (NCHUNK - 1) % 2)

    return k


_GATHER = _sc_gather_concat()


def kernel(word_inputs, feature_inputs_0, feature_inputs_1, word_seq_lengths,
           char_inputs, char_seq_lengths, char_seq_recover,
           word_table, feat_table_0, feat_table_1):
    widx = jnp.reshape(word_inputs.astype(jnp.int32), (NW, NCHUNK, CHUNK))
    f0idx = jnp.reshape(feature_inputs_0.astype(jnp.int32), (NW, NCHUNK, CHUNK))
    f1idx = jnp.reshape(feature_inputs_1.astype(jnp.int32), (NW, NCHUNK, CHUNK))
    out = _GATHER(widx, f0idx, f1idx, word_table, feat_table_0, feat_table_1)
    return jnp.reshape(out, (B, L, OUT_D))
